# baseline (speedup 1.0000x reference)
import functools

import jax
import jax.numpy as jnp
from jax import lax
from jax.experimental import pallas as pl
from jax.experimental.pallas import tpu as pltpu
from jax.experimental.pallas import tpu_sc as plsc

NC, NS, L = 2, 16, 16
NW = NC * NS
ROWS, COLS, D = 16384, 1024, 256
K = 8
RB = 8
NCHUNK = COLS // L
RPW = ROWS // NW
NBLK = RPW // RB
GIDX = RB * K
IDXPAD = GIDX + K

_mesh = plsc.VectorSubcoreMesh(core_axis_name="c", subcore_axis_name="s")


@functools.partial(
    pl.kernel,
    out_type=jax.ShapeDtypeStruct((ROWS, D), jnp.float32),
    mesh=_mesh,
    scratch_types=[
        pltpu.VMEM((RB, COLS), jnp.float32),
        pltpu.VMEM((RB, COLS), jnp.float32),
        pltpu.VMEM((IDXPAD,), jnp.int32),
        pltpu.VMEM((IDXPAD,), jnp.int32),
        pltpu.VMEM((IDXPAD, D), jnp.float32),
        pltpu.VMEM((IDXPAD, D), jnp.float32),
        pltpu.VMEM((RB, D), jnp.float32),
        pltpu.VMEM((RB, D), jnp.float32),
        pltpu.SemaphoreType.DMA,
        pltpu.SemaphoreType.DMA,
        pltpu.SemaphoreType.DMA,
        pltpu.SemaphoreType.DMA,
        pltpu.SemaphoreType.DMA,
        pltpu.SemaphoreType.DMA,
    ],
    compiler_params=pltpu.CompilerParams(needs_layout_passes=False),
)
def _nkq_sc(x_hbm, wt_hbm, out_hbm, xv0, xv1, ix0, ix1, rv0, rv1, ov0, ov1,
            xs0, xs1, gs0, gs1, os0, os1):
    x_v = (xv0, xv1)
    idx_v = (ix0, ix1)
    rows_v = (rv0, rv1)
    out_v = (ov0, ov1)
    xsem = (xs0, xs1)
    gsem = (gs0, gs1)
    osem = (os0, os1)

    wid = lax.axis_index("s") * NC + lax.axis_index("c")
    base0 = wid * RPW
    lanes = lax.iota(jnp.int32, L)
    store_mask = lanes < K
    neg_inf = jnp.full((L,), -jnp.inf, dtype=jnp.float32)
    zeros_i = jnp.zeros((L,), dtype=jnp.int32)

    for p in range(2):
        idx_v[p][pl.ds(IDXPAD - L, L)] = zeros_i

    def start_x(b, p):
        pltpu.async_copy(
            x_hbm.at[pl.ds(base0 + b * RB, RB)], x_v[p], xsem[p])

    def wait_x(b, p):
        pltpu.make_async_copy(
            x_hbm.at[pl.ds(base0 + b * RB, RB)], x_v[p], xsem[p]).wait()

    GSPLIT = ((0, 16), (16, 16), (32, 16), (48, 24))

    def start_g(p):
        for (o, n) in GSPLIT:
            pltpu.async_copy(
                wt_hbm.at[idx_v[p].at[pl.ds(o, n)]],
                rows_v[p].at[pl.ds(o, n)], gsem[p])

    def wait_g(p):
        for (o, n) in GSPLIT:
            pltpu.make_async_copy(
                wt_hbm.at[idx_v[p].at[pl.ds(o, n)]],
                rows_v[p].at[pl.ds(o, n)], gsem[p]).wait()

    def start_o(b, p):
        pltpu.async_copy(
            out_v[p], out_hbm.at[pl.ds(base0 + b * RB, RB)], osem[p])

    def wait_o(b, p):
        pltpu.make_async_copy(
            out_v[p], out_hbm.at[pl.ds(base0 + b * RB, RB)], osem[p]).wait()

    def topk(p):
        def chunk_body(c, st):
            colv = lanes + c * L
            new = []
            for r in range(RB):
                rk, rv = st[2 * r], st[2 * r + 1]
                ck = x_v[p][r, pl.ds(c * L, L)]
                sk, sv = plsc.sort_key_val(ck, colv, descending=False)
                m = rk >= sk
                mk = jnp.where(m, rk, sk)
                mv = jnp.where(m, rv, sv)
                rk, rv = plsc.sort_key_val(mk, mv, descending=True)
                new += [rk, rv]
            return tuple(new)

        init = (neg_inf, zeros_i) * RB
        fin = lax.fori_loop(0, NCHUNK, chunk_body, init)
        for r in range(RB):
            plsc.store_compressed(
                idx_v[p].at[pl.ds(r * K, L)], fin[2 * r + 1], mask=store_mask)

    def accumulate(p):
        def acc_body(j, a):
            for r in range(RB):
                s = rows_v[p][r * K, pl.ds(j * L, L)]
                for k in range(1, K):
                    s = s + rows_v[p][r * K + k, pl.ds(j * L, L)]
                out_v[p][r, pl.ds(j * L, L)] = s
            return a

        lax.fori_loop(0, D // L, acc_body, 0)

    def phase_a(b, p, prefetch):
        wait_x(b, p)
        topk(p)
        start_g(p)
        if prefetch:
            start_x(b + 2, p)

    def phase_b(b, p, wait_out):
        if wait_out:
            wait_o(b - 2, p)
        wait_g(p)
        accumulate(p)
        start_o(b, p)

    start_x(0, 0)
    start_x(1, 1)
    phase_a(0, 0, True)
    phase_a(1, 1, True)
    phase_b(0, 0, False)
    phase_a(2, 0, True)
    phase_b(1, 1, False)

    def main_body(u, carry):
        b1 = 3 + 2 * u
        phase_a(b1, 1, True)
        phase_b(b1 - 1, 0, True)
        phase_a(b1 + 1, 0, True)
        phase_b(b1, 1, True)
        return carry

    lax.fori_loop(0, 29, main_body, 0)

    phase_a(61, 1, True)
    phase_b(60, 0, True)
    phase_a(62, 0, False)
    phase_b(61, 1, True)
    phase_a(63, 1, False)
    phase_b(62, 0, True)
    phase_b(63, 1, True)
    wait_o(62, 0)
    wait_o(63, 1)


def kernel(x, W):
    return _nkq_sc(x, W.T)


# speedup vs baseline: 3.4071x; 3.4071x over previous
import functools

import jax
import jax.numpy as jnp
from jax import lax
from jax.experimental import pallas as pl
from jax.experimental.pallas import tpu as pltpu
from jax.experimental.pallas import tpu_sc as plsc

NC, NS, L = 2, 16, 16
NW = NC * NS
ROWS, COLS, D = 16384, 1024, 256
K = 8
RB = 8
NCHUNK = COLS // L
RPW = ROWS // NW
NBLK = RPW // RB
IDXPAD = RB * K + K
NQ = 4
DQ = D // NQ
TPG = ROWS // NC // (NS // NQ)
TC_ = 128
NTC = TPG // TC_
RPS = ROWS // NC

_mesh = plsc.VectorSubcoreMesh(core_axis_name="c", subcore_axis_name="s")


@functools.partial(
    pl.kernel,
    out_type=(jax.ShapeDtypeStruct((ROWS, D), jnp.float32),
              jax.ShapeDtypeStruct((ROWS * K,), jnp.int32)),
    mesh=_mesh,
    scratch_types=[
        pltpu.VMEM((COLS, DQ), jnp.float32),
        pltpu.VMEM((RB, COLS), jnp.float32),
        pltpu.VMEM((RB, COLS), jnp.float32),
        pltpu.VMEM((IDXPAD,), jnp.int32),
        pltpu.VMEM((TC_ * K,), jnp.int32),
        pltpu.VMEM((TC_, DQ), jnp.float32),
        pltpu.SemaphoreType.DMA,
        pltpu.SemaphoreType.DMA,
        pltpu.SemaphoreType.DMA,
    ],
    compiler_params=pltpu.CompilerParams(
        needs_layout_passes=False, internal_scratch_in_bytes=32768,
        use_tc_tiling_on_sc=False),
)
def _nkq_sc(x_hbm, wt4_hbm, out_hbm, idx_hbm, tab_v, xv0, xv1, idx_v,
            idxc_v, outc_v, xs0, xs1, tsem):
    x_v = (xv0, xv1)
    xsem = (xs0, xs1)

    c = lax.axis_index("c")
    s = lax.axis_index("s")
    row0 = c * RPS + s * RPW
    q = s % NQ
    g = s // NQ
    tok0 = c * RPS + g * TPG
    lanes = lax.iota(jnp.int32, L)
    store_mask = lanes < K
    neg_inf = jnp.full((L,), -jnp.inf, dtype=jnp.float32)
    zeros_i = jnp.zeros((L,), dtype=jnp.int32)

    pltpu.async_copy(wt4_hbm.at[q], tab_v, tsem)

    def start_x(b, p):
        pltpu.async_copy(x_hbm.at[pl.ds(row0 + b * RB, RB)], x_v[p], xsem[p])

    def wait_x(b, p):
        pltpu.make_async_copy(
            x_hbm.at[pl.ds(row0 + b * RB, RB)], x_v[p], xsem[p]).wait()

    def topk(b, p):
        def chunk_body(ci, st):
            colv = lanes + ci * L
            new = []
            for r in range(RB):
                rk, rv = st[2 * r], st[2 * r + 1]
                ck = x_v[p][r, pl.ds(ci * L, L)]
                sk, sv = plsc.sort_key_val(ck, colv, descending=False)
                m = rk >= sk
                mk = jnp.where(m, rk, sk)
                mv = jnp.where(m, rv, sv)
                rk, rv = plsc.sort_key_val(mk, mv, descending=True)
                new += [rk, rv]
            return tuple(new)

        init = (neg_inf, zeros_i) * RB
        fin = lax.fori_loop(0, NCHUNK, chunk_body, init)
        for r in range(RB):
            plsc.store_compressed(
                idx_v.at[pl.ds(r * K, L)], fin[2 * r + 1], mask=store_mask)
        pltpu.sync_copy(
            idx_v.at[pl.ds(0, RB * K)],
            idx_hbm.at[pl.ds((row0 + b * RB) * K, RB * K)])

    start_x(0, 0)
    start_x(1, 1)

    def blk_pair(u, carry):
        b = 2 * u
        wait_x(b, 0)
        topk(b, 0)

        @pl.when(b + 2 < NBLK)
        def _():
            start_x(b + 2, 0)

        wait_x(b + 1, 1)
        topk(b + 1, 1)

        @pl.when(b + 3 < NBLK)
        def _():
            start_x(b + 3, 1)
        return carry

    lax.fori_loop(0, NBLK // 2, blk_pair, 0)

    plsc.subcore_barrier()
    pltpu.make_async_copy(wt4_hbm.at[q], tab_v, tsem).wait()

    def chunk_fn(i, carry):
        pltpu.sync_copy(
            idx_hbm.at[pl.ds((tok0 + i * TC_) * K, TC_ * K)], idxc_v)

        def pair_fn(t2, carry2):
            iv = idxc_v[pl.ds(t2 * 2 * K, L)]
            for h in range(2):
                accs = []
                for j in range(DQ // L):
                    acc = tab_v[iv[h * K], pl.ds(j * L, L)]
                    accs.append(acc)
                for k in range(1, K):
                    e = iv[h * K + k]
                    for j in range(DQ // L):
                        accs[j] = accs[j] + tab_v[e, pl.ds(j * L, L)]
                for j in range(DQ // L):
                    outc_v[t2 * 2 + h, pl.ds(j * L, L)] = accs[j]
            return carry2

        lax.fori_loop(0, TC_ // 2, pair_fn, 0)
        pltpu.sync_copy(
            outc_v,
            out_hbm.at[pl.ds(tok0 + i * TC_, TC_), pl.ds(q * DQ, DQ)])
        return carry

    lax.fori_loop(0, NTC, chunk_fn, 0)


def _prep_body(w_ref, o_ref):
    for qq in range(NQ):
        o_ref[qq] = jnp.transpose(w_ref[pl.ds(qq * DQ, DQ), :], (1, 0))


_prep = pl.pallas_call(
    _prep_body,
    out_shape=jax.ShapeDtypeStruct((NQ, COLS, DQ), jnp.float32),
)


def kernel(x, W):
    wt4 = _prep(W)
    out, _unused_idx = _nkq_sc(x, wt4)
    return out


# speedup vs baseline: 3.4312x; 1.0071x over previous
import functools

import jax
import jax.numpy as jnp
from jax import lax
from jax.experimental import pallas as pl
from jax.experimental.pallas import tpu as pltpu
from jax.experimental.pallas import tpu_sc as plsc

NC, NS, L = 2, 16, 16
NW = NC * NS
ROWS, COLS, D = 16384, 1024, 256
K = 8
RB = 8
NCHUNK = COLS // L
RPW = ROWS // NW
NBLK = RPW // RB
IDXPAD = RB * K + K
NQ = 4
DQ = D // NQ
TPG = ROWS // NC // (NS // NQ)
TC_ = 128
NTC = TPG // TC_
RPS = ROWS // NC

_mesh = plsc.VectorSubcoreMesh(core_axis_name="c", subcore_axis_name="s")


@functools.partial(
    pl.kernel,
    out_type=(jax.ShapeDtypeStruct((ROWS, D), jnp.float32),
              jax.ShapeDtypeStruct((ROWS * K,), jnp.int32)),
    mesh=_mesh,
    scratch_types=[
        pltpu.VMEM((COLS, DQ), jnp.float32),
        pltpu.VMEM((RB, COLS), jnp.float32),
        pltpu.VMEM((RB, COLS), jnp.float32),
        pltpu.VMEM((2 * RB * K + L,), jnp.int32),
        pltpu.VMEM((TC_ * K,), jnp.int32),
        pltpu.VMEM((TC_, DQ), jnp.float32),
        pltpu.SemaphoreType.DMA,
        pltpu.SemaphoreType.DMA,
        pltpu.SemaphoreType.DMA,
    ],
    compiler_params=pltpu.CompilerParams(
        needs_layout_passes=False, internal_scratch_in_bytes=32768,
        use_tc_tiling_on_sc=False),
)
def _nkq_sc(x_hbm, wt4_hbm, out_hbm, idx_hbm, tab_v, xv0, xv1, idx_v,
            idxc_v, outc_v, xs0, xs1, tsem):
    x_v = (xv0, xv1)
    xsem = (xs0, xs1)

    c = lax.axis_index("c")
    s = lax.axis_index("s")
    row0 = c * RPS + s * RPW
    q = s % NQ
    g = s // NQ
    tok0 = c * RPS + g * TPG
    lanes = lax.iota(jnp.int32, L)
    store_mask = lanes < K
    neg_inf = jnp.full((L,), -jnp.inf, dtype=jnp.float32)
    zeros_i = jnp.zeros((L,), dtype=jnp.int32)

    pltpu.async_copy(wt4_hbm.at[q], tab_v, tsem)

    def start_x(b, p):
        pltpu.async_copy(x_hbm.at[pl.ds(row0 + b * RB, RB)], x_v[p], xsem[p])

    def wait_x(b, p):
        pltpu.make_async_copy(
            x_hbm.at[pl.ds(row0 + b * RB, RB)], x_v[p], xsem[p]).wait()

    def topk(b, p, off):
        def chunk_body(ci, st):
            colv = lanes + ci * L
            new = []
            for r in range(RB):
                rk, rv = st[2 * r], st[2 * r + 1]
                ck = x_v[p][r, pl.ds(ci * L, L)]
                sk, sv = plsc.sort_key_val(ck, colv, descending=False)
                m = rk >= sk
                mk = jnp.where(m, rk, sk)
                mv = jnp.where(m, rv, sv)
                rk, rv = plsc.sort_key_val(mk, mv, descending=True)
                new += [rk, rv]
            return tuple(new)

        init = (neg_inf, zeros_i) * RB
        fin = lax.fori_loop(0, NCHUNK, chunk_body, init)
        for r in range(RB):
            plsc.store_compressed(
                idx_v.at[pl.ds(off + r * K, L)], fin[2 * r + 1],
                mask=store_mask)

    start_x(0, 0)
    start_x(1, 1)

    def blk_pair(u, carry):
        b = 2 * u
        wait_x(b, 0)
        topk(b, 0, 0)

        @pl.when(b + 2 < NBLK)
        def _():
            start_x(b + 2, 0)

        wait_x(b + 1, 1)
        topk(b + 1, 1, RB * K)

        @pl.when(b + 3 < NBLK)
        def _():
            start_x(b + 3, 1)

        pltpu.sync_copy(
            idx_v.at[pl.ds(0, 2 * RB * K)],
            idx_hbm.at[pl.ds((row0 + b * RB) * K, 2 * RB * K)])
        return carry

    lax.fori_loop(0, NBLK // 2, blk_pair, 0)

    plsc.subcore_barrier()
    pltpu.make_async_copy(wt4_hbm.at[q], tab_v, tsem).wait()

    def chunk_fn(i, carry):
        pltpu.sync_copy(
            idx_hbm.at[pl.ds((tok0 + i * TC_) * K, TC_ * K)], idxc_v)

        def pair_fn(t2, carry2):
            iv = idxc_v[pl.ds(t2 * 2 * K, L)]
            for h in range(2):
                accs = []
                for j in range(DQ // L):
                    acc = tab_v[iv[h * K], pl.ds(j * L, L)]
                    accs.append(acc)
                for k in range(1, K):
                    e = iv[h * K + k]
                    for j in range(DQ // L):
                        accs[j] = accs[j] + tab_v[e, pl.ds(j * L, L)]
                for j in range(DQ // L):
                    outc_v[t2 * 2 + h, pl.ds(j * L, L)] = accs[j]
            return carry2

        lax.fori_loop(0, TC_ // 2, pair_fn, 0)
        pltpu.sync_copy(
            outc_v,
            out_hbm.at[pl.ds(tok0 + i * TC_, TC_), pl.ds(q * DQ, DQ)])
        return carry

    lax.fori_loop(0, NTC, chunk_fn, 0)


def _prep_body(w_ref, o_ref):
    for qq in range(NQ):
        o_ref[qq] = jnp.transpose(w_ref[pl.ds(qq * DQ, DQ), :], (1, 0))


_prep = pl.pallas_call(
    _prep_body,
    out_shape=jax.ShapeDtypeStruct((NQ, COLS, DQ), jnp.float32),
)


def kernel(x, W):
    wt4 = _prep(W)
    out, _unused_idx = _nkq_sc(x, wt4)
    return out


# speedup vs baseline: 3.6757x; 1.0713x over previous
import functools

import jax
import jax.numpy as jnp
from jax import lax
from jax.experimental import pallas as pl
from jax.experimental.pallas import tpu as pltpu
from jax.experimental.pallas import tpu_sc as plsc

NC, NS, L = 2, 16, 16
NW = NC * NS
ROWS, COLS, D = 16384, 1024, 256
K = 8
RB = 8
NCHUNK = COLS // L
RPW = ROWS // NW
NBLK = RPW // RB
IDXPAD = RB * K + K
NQ = 4
DQ = D // NQ
TPG = ROWS // NC // (NS // NQ)
TC_ = 128
NTC = TPG // TC_
RPS = ROWS // NC

_mesh = plsc.VectorSubcoreMesh(core_axis_name="c", subcore_axis_name="s")


@functools.partial(
    pl.kernel,
    out_type=(jax.ShapeDtypeStruct((NQ * ROWS * DQ,), jnp.float32),
              jax.ShapeDtypeStruct((ROWS * K,), jnp.int32)),
    mesh=_mesh,
    scratch_types=[
        pltpu.VMEM((COLS * DQ,), jnp.float32),
        pltpu.VMEM((RB, COLS), jnp.float32),
        pltpu.VMEM((RB, COLS), jnp.float32),
        pltpu.VMEM((2 * RB * K + L,), jnp.int32),
        pltpu.VMEM((TC_ * K,), jnp.int32),
        pltpu.VMEM((TC_ * DQ,), jnp.float32),
        pltpu.SemaphoreType.DMA,
        pltpu.SemaphoreType.DMA,
        pltpu.SemaphoreType.DMA,
    ],
    compiler_params=pltpu.CompilerParams(
        needs_layout_passes=False, internal_scratch_in_bytes=32768),
)
def _nkq_sc(x_hbm, wt4_hbm, out_hbm, idx_hbm, tab_v, xv0, xv1, idx_v,
            idxc_v, outc_v, xs0, xs1, tsem):
    x_v = (xv0, xv1)
    xsem = (xs0, xs1)

    c = lax.axis_index("c")
    s = lax.axis_index("s")
    row0 = c * RPS + s * RPW
    q = s % NQ
    g = s // NQ
    tok0 = c * RPS + g * TPG
    lanes = lax.iota(jnp.int32, L)
    store_mask = lanes < K
    neg_inf = jnp.full((L,), -jnp.inf, dtype=jnp.float32)
    zeros_i = jnp.zeros((L,), dtype=jnp.int32)

    pltpu.async_copy(wt4_hbm.at[pl.ds(q * (COLS * DQ), COLS * DQ)], tab_v,
                     tsem)

    def start_x(b, p):
        pltpu.async_copy(x_hbm.at[pl.ds(row0 + b * RB, RB)], x_v[p], xsem[p])

    def wait_x(b, p):
        pltpu.make_async_copy(
            x_hbm.at[pl.ds(row0 + b * RB, RB)], x_v[p], xsem[p]).wait()

    def topk(b, p, off):
        def chunk_body(ci, st):
            colv = lanes + ci * L
            new = []
            for r in range(RB):
                rk, rv = st[2 * r], st[2 * r + 1]
                ck = x_v[p][r, pl.ds(ci * L, L)]
                sk, sv = plsc.sort_key_val(ck, colv, descending=False)
                m = rk >= sk
                mk = jnp.where(m, rk, sk)
                mv = jnp.where(m, rv, sv)
                rk, rv = plsc.sort_key_val(mk, mv, descending=True)
                new += [rk, rv]
            return tuple(new)

        init = (neg_inf, zeros_i) * RB
        fin = lax.fori_loop(0, NCHUNK, chunk_body, init)
        for r in range(RB):
            plsc.store_compressed(
                idx_v.at[pl.ds(off + r * K, L)], fin[2 * r + 1],
                mask=store_mask)

    start_x(0, 0)
    start_x(1, 1)

    def blk_pair(u, carry):
        b = 2 * u
        wait_x(b, 0)
        topk(b, 0, 0)

        @pl.when(b + 2 < NBLK)
        def _():
            start_x(b + 2, 0)

        wait_x(b + 1, 1)
        topk(b + 1, 1, RB * K)

        @pl.when(b + 3 < NBLK)
        def _():
            start_x(b + 3, 1)

        pltpu.sync_copy(
            idx_v.at[pl.ds(0, 2 * RB * K)],
            idx_hbm.at[pl.ds((row0 + b * RB) * K, 2 * RB * K)])
        return carry

    lax.fori_loop(0, NBLK // 2, blk_pair, 0)

    plsc.subcore_barrier()
    pltpu.make_async_copy(
        wt4_hbm.at[pl.ds(q * (COLS * DQ), COLS * DQ)], tab_v, tsem).wait()

    def chunk_fn(i, carry):
        pltpu.sync_copy(
            idx_hbm.at[pl.ds((tok0 + i * TC_) * K, TC_ * K)], idxc_v)

        def pair_fn(t2, carry2):
            iv = idxc_v[pl.ds(t2 * 2 * K, L)]
            for h in range(2):
                e0 = iv[h * K] * DQ
                accs = []
                for j in range(DQ // L):
                    accs.append(tab_v[pl.ds(e0 + j * L, L)])
                for k in range(1, K):
                    e = iv[h * K + k] * DQ
                    for j in range(DQ // L):
                        accs[j] = accs[j] + tab_v[pl.ds(e + j * L, L)]
                for j in range(DQ // L):
                    outc_v[pl.ds((t2 * 2 + h) * DQ + j * L, L)] = accs[j]
            return carry2

        lax.fori_loop(0, TC_ // 2, pair_fn, 0)
        pltpu.sync_copy(
            outc_v,
            out_hbm.at[pl.ds(q * (ROWS * DQ) + (tok0 + i * TC_) * DQ,
                             TC_ * DQ)])
        return carry

    lax.fori_loop(0, NTC, chunk_fn, 0)


def _prep_body(w_ref, o_ref):
    for qq in range(NQ):
        o_ref[qq] = jnp.transpose(w_ref[pl.ds(qq * DQ, DQ), :], (1, 0))


_prep = pl.pallas_call(
    _prep_body,
    out_shape=jax.ShapeDtypeStruct((NQ, COLS, DQ), jnp.float32),
)


def _asm_body(o4_ref, o_ref):
    o_ref[...] = jnp.concatenate(
        [o4_ref[qq] for qq in range(NQ)], axis=-1)


_assemble = pl.pallas_call(
    _asm_body,
    grid=(16,),
    in_specs=[pl.BlockSpec((NQ, ROWS // 16, DQ), lambda i: (0, i, 0))],
    out_specs=pl.BlockSpec((ROWS // 16, D), lambda i: (i, 0)),
    out_shape=jax.ShapeDtypeStruct((ROWS, D), jnp.float32),
)


def kernel(x, W):
    wt4 = _prep(W).reshape(NQ * COLS * DQ)
    out4_flat, _unused_idx = _nkq_sc(x, wt4)
    return _assemble(out4_flat.reshape(NQ, ROWS, DQ))


# speedup vs baseline: 3.9823x; 1.0834x over previous
import functools

import jax
import jax.numpy as jnp
from jax import lax
from jax.experimental import pallas as pl
from jax.experimental.pallas import tpu as pltpu
from jax.experimental.pallas import tpu_sc as plsc

NC, NS, L = 2, 16, 16
NW = NC * NS
ROWS, COLS, D = 16384, 1024, 256
K = 8
RB = 8
NCHUNK = COLS // L
RPW = ROWS // NW
NBLK = RPW // RB
IDXPAD = RB * K + K
NQ = 4
DQ = D // NQ
TPG = ROWS // NC // (NS // NQ)
TC_ = 128
NTC = TPG // TC_
RPS = ROWS // NC

_mesh = plsc.VectorSubcoreMesh(core_axis_name="c", subcore_axis_name="s")


@functools.partial(
    pl.kernel,
    out_type=(jax.ShapeDtypeStruct((NQ * ROWS * DQ,), jnp.float32),
              jax.ShapeDtypeStruct((ROWS * K,), jnp.int32)),
    mesh=_mesh,
    scratch_types=[
        pltpu.VMEM((COLS * DQ,), jnp.float32),
        pltpu.VMEM((RB, COLS), jnp.float32),
        pltpu.VMEM((RB, COLS), jnp.float32),
        pltpu.VMEM((2 * RB * K + L,), jnp.int32),
        pltpu.VMEM((TC_ * K,), jnp.int32),
        pltpu.VMEM((TC_ * K,), jnp.int32),
        pltpu.VMEM((TC_ * DQ,), jnp.float32),
        pltpu.VMEM((TC_ * DQ,), jnp.float32),
        pltpu.SemaphoreType.DMA,
        pltpu.SemaphoreType.DMA,
        pltpu.SemaphoreType.DMA,
        pltpu.SemaphoreType.DMA,
        pltpu.SemaphoreType.DMA,
        pltpu.SemaphoreType.DMA,
        pltpu.SemaphoreType.DMA,
    ],
    compiler_params=pltpu.CompilerParams(
        needs_layout_passes=False, internal_scratch_in_bytes=32768),
)
def _nkq_sc(x_hbm, wt4_hbm, out_hbm, idx_hbm, tab_v, xv0, xv1, idx_v,
            ic0, ic1, oc0, oc1, xs0, xs1, tsem, is0, is1, os0, os1):
    x_v = (xv0, xv1)
    xsem = (xs0, xs1)
    idxc_v = (ic0, ic1)
    outc_v = (oc0, oc1)
    isem = (is0, is1)
    osem = (os0, os1)

    c = lax.axis_index("c")
    s = lax.axis_index("s")
    row0 = c * RPS + s * RPW
    q = s % NQ
    g = s // NQ
    tok0 = c * RPS + g * TPG
    lanes = lax.iota(jnp.int32, L)
    store_mask = lanes < K
    neg_inf = jnp.full((L,), -jnp.inf, dtype=jnp.float32)
    zeros_i = jnp.zeros((L,), dtype=jnp.int32)

    pltpu.async_copy(wt4_hbm.at[pl.ds(q * (COLS * DQ), COLS * DQ)], tab_v,
                     tsem)

    def start_x(b, p):
        pltpu.async_copy(x_hbm.at[pl.ds(row0 + b * RB, RB)], x_v[p], xsem[p])

    def wait_x(b, p):
        pltpu.make_async_copy(
            x_hbm.at[pl.ds(row0 + b * RB, RB)], x_v[p], xsem[p]).wait()

    def topk(b, p, off):
        def chunk_body(ci, st):
            colv = lanes + ci * L
            new = []
            for r in range(RB):
                rk, rv = st[2 * r], st[2 * r + 1]
                ck = x_v[p][r, pl.ds(ci * L, L)]
                sk, sv = plsc.sort_key_val(ck, colv, descending=False)
                m = rk >= sk
                mk = jnp.where(m, rk, sk)
                mv = jnp.where(m, rv, sv)
                rk, rv = plsc.sort_key_val(mk, mv, descending=True)
                new += [rk, rv]
            return tuple(new)

        init = (neg_inf, zeros_i) * RB
        fin = lax.fori_loop(0, NCHUNK, chunk_body, init)
        for r in range(RB):
            plsc.store_compressed(
                idx_v.at[pl.ds(off + r * K, L)], fin[2 * r + 1],
                mask=store_mask)

    start_x(0, 0)
    start_x(1, 1)

    def blk_pair(u, carry):
        b = 2 * u
        wait_x(b, 0)
        topk(b, 0, 0)

        @pl.when(b + 2 < NBLK)
        def _():
            start_x(b + 2, 0)

        wait_x(b + 1, 1)
        topk(b + 1, 1, RB * K)

        @pl.when(b + 3 < NBLK)
        def _():
            start_x(b + 3, 1)

        pltpu.sync_copy(
            idx_v.at[pl.ds(0, 2 * RB * K)],
            idx_hbm.at[pl.ds((row0 + b * RB) * K, 2 * RB * K)])
        return carry

    lax.fori_loop(0, NBLK // 2, blk_pair, 0)

    plsc.subcore_barrier()
    pltpu.make_async_copy(
        wt4_hbm.at[pl.ds(q * (COLS * DQ), COLS * DQ)], tab_v, tsem).wait()

    def load_c(i, p):
        pltpu.async_copy(
            idx_hbm.at[pl.ds((tok0 + i * TC_) * K, TC_ * K)], idxc_v[p],
            isem[p])

    def wait_load_c(i, p):
        pltpu.make_async_copy(
            idx_hbm.at[pl.ds((tok0 + i * TC_) * K, TC_ * K)], idxc_v[p],
            isem[p]).wait()

    def write_c(i, p):
        pltpu.async_copy(
            outc_v[p],
            out_hbm.at[pl.ds(q * (ROWS * DQ) + (tok0 + i * TC_) * DQ,
                             TC_ * DQ)], osem[p])

    def wait_write_c(i, p):
        pltpu.make_async_copy(
            outc_v[p],
            out_hbm.at[pl.ds(q * (ROWS * DQ) + (tok0 + i * TC_) * DQ,
                             TC_ * DQ)], osem[p]).wait()

    def compute_c(p):
        def pair_fn(t2, carry2):
            iv = idxc_v[p][pl.ds(t2 * 2 * K, L)]
            for h in range(2):
                e0 = iv[h * K] * DQ
                accs = []
                for j in range(DQ // L):
                    accs.append(tab_v[pl.ds(e0 + j * L, L)])
                for k in range(1, K):
                    e = iv[h * K + k] * DQ
                    for j in range(DQ // L):
                        accs[j] = accs[j] + tab_v[pl.ds(e + j * L, L)]
                for j in range(DQ // L):
                    outc_v[p][pl.ds((t2 * 2 + h) * DQ + j * L, L)] = accs[j]
            return carry2

        lax.fori_loop(0, TC_ // 2, pair_fn, 0)

    load_c(0, 0)
    load_c(1, 1)

    def chunk_pair(u, carry):
        for h2 in range(2):
            i = 2 * u + h2
            wait_load_c(i, h2)

            @pl.when(i >= 2)
            def _():
                wait_write_c(i - 2, h2)

            compute_c(h2)
            write_c(i, h2)

            @pl.when(i + 2 < NTC)
            def _():
                load_c(i + 2, h2)
        return carry

    lax.fori_loop(0, NTC // 2, chunk_pair, 0)
    wait_write_c(NTC - 2, 0)
    wait_write_c(NTC - 1, 1)


def _prep_body(w_ref, o_ref):
    for qq in range(NQ):
        o_ref[qq] = jnp.transpose(w_ref[pl.ds(qq * DQ, DQ), :], (1, 0))


_prep = pl.pallas_call(
    _prep_body,
    out_shape=jax.ShapeDtypeStruct((NQ, COLS, DQ), jnp.float32),
)


def _asm_body(o4_ref, o_ref):
    o_ref[...] = jnp.concatenate(
        [o4_ref[qq] for qq in range(NQ)], axis=-1)


_assemble = pl.pallas_call(
    _asm_body,
    grid=(16,),
    in_specs=[pl.BlockSpec((NQ, ROWS // 16, DQ), lambda i: (0, i, 0))],
    out_specs=pl.BlockSpec((ROWS // 16, D), lambda i: (i, 0)),
    out_shape=jax.ShapeDtypeStruct((ROWS, D), jnp.float32),
)


def kernel(x, W):
    wt4 = _prep(W).reshape(NQ * COLS * DQ)
    out4_flat, _unused_idx = _nkq_sc(x, wt4)
    return _assemble(out4_flat.reshape(NQ, ROWS, DQ))


# speedup vs baseline: 4.0149x; 1.0082x over previous
import functools

import jax
import jax.numpy as jnp
from jax import lax
from jax.experimental import pallas as pl
from jax.experimental.pallas import tpu as pltpu
from jax.experimental.pallas import tpu_sc as plsc

NC, NS, L = 2, 16, 16
NW = NC * NS
ROWS, COLS, D = 16384, 1024, 256
K = 8
RB = 8
NCHUNK = COLS // L
RPW = ROWS // NW
NBLK = RPW // RB
IDXPAD = RB * K + K
NQ = 4
DQ = D // NQ
TPG = ROWS // NC // (NS // NQ)
TC_ = 128
NTC = TPG // TC_
RPS = ROWS // NC

_mesh = plsc.VectorSubcoreMesh(core_axis_name="c", subcore_axis_name="s")


@functools.partial(
    pl.kernel,
    out_type=(jax.ShapeDtypeStruct((NQ * ROWS * DQ,), jnp.float32),
              jax.ShapeDtypeStruct((ROWS * K,), jnp.int32)),
    mesh=_mesh,
    scratch_types=[
        pltpu.VMEM((COLS * DQ,), jnp.float32),
        pltpu.VMEM((RB, COLS), jnp.float32),
        pltpu.VMEM((RB, COLS), jnp.float32),
        pltpu.VMEM((2 * RB * K + L,), jnp.int32),
        pltpu.VMEM((TC_ * K,), jnp.int32),
        pltpu.VMEM((TC_ * K,), jnp.int32),
        pltpu.VMEM((TC_ * DQ,), jnp.float32),
        pltpu.VMEM((TC_ * DQ,), jnp.float32),
        pltpu.SemaphoreType.DMA,
        pltpu.SemaphoreType.DMA,
        pltpu.SemaphoreType.DMA,
        pltpu.SemaphoreType.DMA,
        pltpu.SemaphoreType.DMA,
        pltpu.SemaphoreType.DMA,
        pltpu.SemaphoreType.DMA,
    ],
    compiler_params=pltpu.CompilerParams(
        needs_layout_passes=False, internal_scratch_in_bytes=32768),
)
def _nkq_sc(x_hbm, wt4_hbm, out_hbm, idx_hbm, tab_v, xv0, xv1, idx_v,
            ic0, ic1, oc0, oc1, xs0, xs1, tsem, is0, is1, os0, os1):
    x_v = (xv0, xv1)
    xsem = (xs0, xs1)
    idxc_v = (ic0, ic1)
    outc_v = (oc0, oc1)
    isem = (is0, is1)
    osem = (os0, os1)

    c = lax.axis_index("c")
    s = lax.axis_index("s")
    row0 = c * RPS + s * RPW
    q = s % NQ
    g = s // NQ
    tok0 = c * RPS + g * TPG
    lanes = lax.iota(jnp.int32, L)
    store_mask = lanes < K
    neg_inf = jnp.full((L,), -jnp.inf, dtype=jnp.float32)
    zeros_i = jnp.zeros((L,), dtype=jnp.int32)

    pltpu.async_copy(wt4_hbm.at[pl.ds(q * (COLS * DQ), COLS * DQ)], tab_v,
                     tsem)

    def start_x(b, p):
        pltpu.async_copy(x_hbm.at[pl.ds(row0 + b * RB, RB)], x_v[p], xsem[p])

    def wait_x(b, p):
        pltpu.make_async_copy(
            x_hbm.at[pl.ds(row0 + b * RB, RB)], x_v[p], xsem[p]).wait()

    def topk(b, p, off):
        def chunk_body(ci, st):
            colv = lanes + ci * L
            new = []
            for r in range(RB):
                rk, rv = st[2 * r], st[2 * r + 1]
                ck = x_v[p][r, pl.ds(ci * L, L)]
                sk, sv = plsc.sort_key_val(ck, colv, descending=False)
                m = rk >= sk
                mk = jnp.where(m, rk, sk)
                mv = jnp.where(m, rv, sv)
                rk, rv = plsc.sort_key_val(mk, mv, descending=True)
                new += [rk, rv]
            return tuple(new)

        init = (neg_inf, zeros_i) * RB
        fin = lax.fori_loop(0, NCHUNK, chunk_body, init)
        for r in range(RB):
            plsc.store_compressed(
                idx_v.at[pl.ds(off + r * K, L)], fin[2 * r + 1],
                mask=store_mask)

    start_x(0, 0)
    start_x(1, 1)

    def blk_pair(u, carry):
        b = 2 * u
        wait_x(b, 0)
        topk(b, 0, 0)

        @pl.when(b + 2 < NBLK)
        def _():
            start_x(b + 2, 0)

        wait_x(b + 1, 1)
        topk(b + 1, 1, RB * K)

        @pl.when(b + 3 < NBLK)
        def _():
            start_x(b + 3, 1)

        pltpu.sync_copy(
            idx_v.at[pl.ds(0, 2 * RB * K)],
            idx_hbm.at[pl.ds((row0 + b * RB) * K, 2 * RB * K)])
        return carry

    lax.fori_loop(0, NBLK // 2, blk_pair, 0)

    plsc.subcore_barrier()
    pltpu.make_async_copy(
        wt4_hbm.at[pl.ds(q * (COLS * DQ), COLS * DQ)], tab_v, tsem).wait()

    def load_c(i, p):
        pltpu.async_copy(
            idx_hbm.at[pl.ds((tok0 + i * TC_) * K, TC_ * K)], idxc_v[p],
            isem[p])

    def wait_load_c(i, p):
        pltpu.make_async_copy(
            idx_hbm.at[pl.ds((tok0 + i * TC_) * K, TC_ * K)], idxc_v[p],
            isem[p]).wait()

    def write_c(i, p):
        pltpu.async_copy(
            outc_v[p],
            out_hbm.at[pl.ds(q * (ROWS * DQ) + (tok0 + i * TC_) * DQ,
                             TC_ * DQ)], osem[p])

    def wait_write_c(i, p):
        pltpu.make_async_copy(
            outc_v[p],
            out_hbm.at[pl.ds(q * (ROWS * DQ) + (tok0 + i * TC_) * DQ,
                             TC_ * DQ)], osem[p]).wait()

    def compute_c(p):
        def pair_fn(t2, carry2):
            iv = idxc_v[p][pl.ds(t2 * 2 * K, L)]
            for h in range(2):
                e0 = iv[h * K] * DQ
                accs = []
                for j in range(DQ // L):
                    accs.append(tab_v[pl.ds(e0 + j * L, L)])
                for k in range(1, K):
                    e = iv[h * K + k] * DQ
                    for j in range(DQ // L):
                        accs[j] = accs[j] + tab_v[pl.ds(e + j * L, L)]
                for j in range(DQ // L):
                    outc_v[p][pl.ds((t2 * 2 + h) * DQ + j * L, L)] = accs[j]
            return carry2

        lax.fori_loop(0, TC_ // 2, pair_fn, 0, unroll=2)

    load_c(0, 0)
    load_c(1, 1)

    def chunk_pair(u, carry):
        for h2 in range(2):
            i = 2 * u + h2
            wait_load_c(i, h2)

            @pl.when(i >= 2)
            def _():
                wait_write_c(i - 2, h2)

            compute_c(h2)
            write_c(i, h2)

            @pl.when(i + 2 < NTC)
            def _():
                load_c(i + 2, h2)
        return carry

    lax.fori_loop(0, NTC // 2, chunk_pair, 0)
    wait_write_c(NTC - 2, 0)
    wait_write_c(NTC - 1, 1)


def _prep_body(w_ref, o_ref):
    for qq in range(NQ):
        o_ref[qq] = jnp.transpose(w_ref[pl.ds(qq * DQ, DQ), :], (1, 0))


_prep = pl.pallas_call(
    _prep_body,
    out_shape=jax.ShapeDtypeStruct((NQ, COLS, DQ), jnp.float32),
)


def _asm_body(o4_ref, o_ref):
    o_ref[...] = jnp.concatenate(
        [o4_ref[qq] for qq in range(NQ)], axis=-1)


_assemble = pl.pallas_call(
    _asm_body,
    grid=(16,),
    in_specs=[pl.BlockSpec((NQ, ROWS // 16, DQ), lambda i: (0, i, 0))],
    out_specs=pl.BlockSpec((ROWS // 16, D), lambda i: (i, 0)),
    out_shape=jax.ShapeDtypeStruct((ROWS, D), jnp.float32),
)


def kernel(x, W):
    wt4 = _prep(W).reshape(NQ * COLS * DQ)
    out4_flat, _unused_idx = _nkq_sc(x, wt4)
    return _assemble(out4_flat.reshape(NQ, ROWS, DQ))
